# baseline (speedup 1.0000x reference)
import functools

import jax
import jax.numpy as jnp
from jax.experimental import pallas as pl
from jax.experimental.pallas import tpu as pltpu

Q = 1024
D = 32
V = 100000
NEG_INV_T_LOG2E = -10.0 * 1.4426950408889634
VBLK = 5000


def _soft_min_kernel(q_ref, k_ref, score_ref, thresh_ref, vocab_ref,
                     qn_ref, colf_ref, m_ref, z_ref, w_ref, idxf_ref,
                     *, nblk, vblk):
    i = pl.program_id(0)

    @pl.when(i == 0)
    def _init():
        q = q_ref[...]
        qnorm = jnp.sqrt(jnp.sum(q * q, axis=-1, keepdims=True))
        qn_ref[...] = q / (qnorm + 1e-8)
        colf_ref[...] = jax.lax.broadcasted_iota(
            jnp.int32, (1, vblk), 1).astype(jnp.float32)
        m_ref[...] = jnp.full((Q, 1), jnp.inf, jnp.float32)
        z_ref[...] = jnp.zeros((Q, 1), jnp.float32)
        w_ref[...] = jnp.zeros((Q, 1), jnp.float32)
        idxf_ref[...] = jnp.zeros((Q, 1), jnp.float32)

    k = k_ref[...]
    knorm = jnp.sqrt(jnp.sum(k * k, axis=-1, keepdims=True))
    kn = k / (knorm + 1e-8)
    sim = jax.lax.dot_general(
        qn_ref[...], kn, (((1,), (1,)), ((), ())),
        preferred_element_type=jnp.float32)
    dist = 1.0 - sim

    e = jnp.exp2(dist * NEG_INV_T_LOG2E)
    z_ref[...] += jnp.sum(e, axis=1, keepdims=True)
    w_ref[...] += jnp.sum(dist * e, axis=1, keepdims=True)

    bm = jnp.min(dist, axis=1, keepdims=True)
    ba = jnp.min(jnp.where(dist <= bm, colf_ref[...], float(vblk)),
                 axis=1, keepdims=True)
    idxf_ref[...] = jnp.where(bm < m_ref[...], ba + i * float(vblk),
                              idxf_ref[...])
    m_ref[...] = jnp.minimum(m_ref[...], bm)

    @pl.when(i == nblk - 1)
    def _finish():
        value = w_ref[...] / z_ref[...]
        score_ref[...] = value
        t = 1.0 - 2.0 * value
        celu = jnp.where(t > 0.0, t, jnp.exp(t) - 1.0)
        thresh_ref[...] = (celu + 1.0) * 0.5
        vocab_ref[...] = idxf_ref[...].astype(jnp.int32)


@jax.jit
def kernel(queries, keys):
    nblk = V // VBLK
    out = pl.pallas_call(
        functools.partial(_soft_min_kernel, nblk=nblk, vblk=VBLK),
        grid=(nblk,),
        in_specs=[
            pl.BlockSpec((Q, D), lambda i: (0, 0)),
            pl.BlockSpec((VBLK, D), lambda i: (i, 0)),
        ],
        out_specs=[
            pl.BlockSpec((Q, 1), lambda i: (0, 0)),
            pl.BlockSpec((Q, 1), lambda i: (0, 0)),
            pl.BlockSpec((Q, 1), lambda i: (0, 0)),
        ],
        out_shape=[
            jax.ShapeDtypeStruct((Q, 1), jnp.float32),
            jax.ShapeDtypeStruct((Q, 1), jnp.float32),
            jax.ShapeDtypeStruct((Q, 1), jnp.int32),
        ],
        scratch_shapes=[
            pltpu.VMEM((Q, D), jnp.float32),
            pltpu.VMEM((1, VBLK), jnp.float32),
            pltpu.VMEM((Q, 1), jnp.float32),
            pltpu.VMEM((Q, 1), jnp.float32),
            pltpu.VMEM((Q, 1), jnp.float32),
            pltpu.VMEM((Q, 1), jnp.float32),
        ],
    )(queries, keys)
    score, thresh, vocab = out
    return score.reshape(-1), thresh.reshape(-1), vocab.reshape(-1)


# speedup vs baseline: 1.0154x; 1.0154x over previous
import functools

import jax
import jax.numpy as jnp
from jax.experimental import pallas as pl
from jax.experimental.pallas import tpu as pltpu

Q = 1024
D = 32
V = 100000
NEG_INV_T_LOG2E = -10.0 * 1.4426950408889634
VBLK = 4000


def _soft_min_kernel(q_ref, k_ref, score_ref, thresh_ref, vocab_ref,
                     qn_ref, colf_ref, m_ref, z_ref, w_ref, idxf_ref,
                     *, nblk, vblk):
    i = pl.program_id(0)

    @pl.when(i == 0)
    def _init():
        q = q_ref[...]
        qnorm = jnp.sqrt(jnp.sum(q * q, axis=-1, keepdims=True))
        qn_ref[...] = q / (qnorm + 1e-8)
        colf_ref[...] = jax.lax.broadcasted_iota(
            jnp.int32, (1, vblk), 1).astype(jnp.float32)
        m_ref[...] = jnp.full((Q, 1), jnp.inf, jnp.float32)
        z_ref[...] = jnp.zeros((Q, 1), jnp.float32)
        w_ref[...] = jnp.zeros((Q, 1), jnp.float32)
        idxf_ref[...] = jnp.zeros((Q, 1), jnp.float32)

    k = k_ref[...]
    knorm = jnp.sqrt(jnp.sum(k * k, axis=-1, keepdims=True))
    kn = k / (knorm + 1e-8)
    sim = jax.lax.dot_general(
        qn_ref[...], kn, (((1,), (1,)), ((), ())),
        preferred_element_type=jnp.float32)
    dist = 1.0 - sim

    e = jnp.exp2(dist * NEG_INV_T_LOG2E)
    z_ref[...] += jnp.sum(e, axis=1, keepdims=True)
    w_ref[...] += jnp.sum(dist * e, axis=1, keepdims=True)

    bm = jnp.min(dist, axis=1, keepdims=True)
    ba = jnp.min(jnp.where(dist <= bm, colf_ref[...], float(vblk)),
                 axis=1, keepdims=True)
    idxf_ref[...] = jnp.where(bm < m_ref[...], ba + i * float(vblk),
                              idxf_ref[...])
    m_ref[...] = jnp.minimum(m_ref[...], bm)

    @pl.when(i == nblk - 1)
    def _finish():
        value = w_ref[...] / z_ref[...]
        score_ref[...] = value
        t = 1.0 - 2.0 * value
        celu = jnp.where(t > 0.0, t, jnp.exp(t) - 1.0)
        thresh_ref[...] = (celu + 1.0) * 0.5
        vocab_ref[...] = idxf_ref[...].astype(jnp.int32)


@jax.jit
def kernel(queries, keys):
    nblk = V // VBLK
    out = pl.pallas_call(
        functools.partial(_soft_min_kernel, nblk=nblk, vblk=VBLK),
        grid=(nblk,),
        in_specs=[
            pl.BlockSpec((Q, D), lambda i: (0, 0)),
            pl.BlockSpec((VBLK, D), lambda i: (i, 0)),
        ],
        out_specs=[
            pl.BlockSpec((Q, 1), lambda i: (0, 0)),
            pl.BlockSpec((Q, 1), lambda i: (0, 0)),
            pl.BlockSpec((Q, 1), lambda i: (0, 0)),
        ],
        out_shape=[
            jax.ShapeDtypeStruct((Q, 1), jnp.float32),
            jax.ShapeDtypeStruct((Q, 1), jnp.float32),
            jax.ShapeDtypeStruct((Q, 1), jnp.int32),
        ],
        scratch_shapes=[
            pltpu.VMEM((Q, D), jnp.float32),
            pltpu.VMEM((1, VBLK), jnp.float32),
            pltpu.VMEM((Q, 1), jnp.float32),
            pltpu.VMEM((Q, 1), jnp.float32),
            pltpu.VMEM((Q, 1), jnp.float32),
            pltpu.VMEM((Q, 1), jnp.float32),
        ],
    )(queries, keys)
    score, thresh, vocab = out
    return score.reshape(-1), thresh.reshape(-1), vocab.reshape(-1)


# speedup vs baseline: 1.0576x; 1.0415x over previous
import functools

import jax
import jax.numpy as jnp
from jax.experimental import pallas as pl
from jax.experimental.pallas import tpu as pltpu

Q = 1024
D = 32
V = 100000
NEG_INV_T_LOG2E = -10.0 * 1.4426950408889634
VBLK = 2000


def _soft_min_kernel(q_ref, k_ref, score_ref, thresh_ref, vocab_ref,
                     qn_ref, colf_ref, macc_ref, z_ref, w_ref, iacc_ref,
                     *, nblk, vblk):
    i = pl.program_id(0)

    @pl.when(i == 0)
    def _init():
        q = q_ref[...]
        qnorm = jnp.sqrt(jnp.sum(q * q, axis=-1, keepdims=True))
        qn_ref[...] = q / (qnorm + 1e-8)
        colf_ref[...] = jax.lax.broadcasted_iota(
            jnp.int32, (1, vblk), 1).astype(jnp.float32)
        macc_ref[...] = jnp.full((Q, vblk), jnp.inf, jnp.float32)
        z_ref[...] = jnp.zeros((Q, 1), jnp.float32)
        w_ref[...] = jnp.zeros((Q, 1), jnp.float32)
        iacc_ref[...] = jnp.zeros((Q, vblk), jnp.float32)

    k = k_ref[...]
    knorm = jnp.sqrt(jnp.sum(k * k, axis=-1, keepdims=True))
    kn = k / (knorm + 1e-8)
    sim = jax.lax.dot_general(
        qn_ref[...], kn, (((1,), (1,)), ((), ())),
        preferred_element_type=jnp.float32)
    dist = 1.0 - sim

    e = jnp.exp2(dist * NEG_INV_T_LOG2E)
    z_ref[...] += jnp.sum(e, axis=1, keepdims=True)
    w_ref[...] += jnp.sum(dist * e, axis=1, keepdims=True)

    cg = colf_ref[...] + (i * vblk).astype(jnp.float32)
    macc = macc_ref[...]
    iacc_ref[...] = jnp.where(dist < macc, cg, iacc_ref[...])
    macc_ref[...] = jnp.minimum(macc, dist)

    @pl.when(i == nblk - 1)
    def _finish():
        value = w_ref[...] / z_ref[...]
        score_ref[...] = value
        t = 1.0 - 2.0 * value
        celu = jnp.where(t > 0.0, t, jnp.exp(t) - 1.0)
        thresh_ref[...] = (celu + 1.0) * 0.5
        macc = macc_ref[...]
        bm = jnp.min(macc, axis=1, keepdims=True)
        cand = jnp.where(macc <= bm, iacc_ref[...], float(V))
        vocab_ref[...] = jnp.min(cand, axis=1,
                                 keepdims=True).astype(jnp.int32)


@jax.jit
def kernel(queries, keys):
    nblk = V // VBLK
    out = pl.pallas_call(
        functools.partial(_soft_min_kernel, nblk=nblk, vblk=VBLK),
        grid=(nblk,),
        in_specs=[
            pl.BlockSpec((Q, D), lambda i: (0, 0)),
            pl.BlockSpec((VBLK, D), lambda i: (i, 0)),
        ],
        out_specs=[
            pl.BlockSpec((Q, 1), lambda i: (0, 0)),
            pl.BlockSpec((Q, 1), lambda i: (0, 0)),
            pl.BlockSpec((Q, 1), lambda i: (0, 0)),
        ],
        out_shape=[
            jax.ShapeDtypeStruct((Q, 1), jnp.float32),
            jax.ShapeDtypeStruct((Q, 1), jnp.float32),
            jax.ShapeDtypeStruct((Q, 1), jnp.int32),
        ],
        scratch_shapes=[
            pltpu.VMEM((Q, D), jnp.float32),
            pltpu.VMEM((1, VBLK), jnp.float32),
            pltpu.VMEM((Q, VBLK), jnp.float32),
            pltpu.VMEM((Q, 1), jnp.float32),
            pltpu.VMEM((Q, 1), jnp.float32),
            pltpu.VMEM((Q, VBLK), jnp.float32),
        ],
    )(queries, keys)
    score, thresh, vocab = out
    return score.reshape(-1), thresh.reshape(-1), vocab.reshape(-1)


# speedup vs baseline: 1.0776x; 1.0189x over previous
import functools

import jax
import jax.numpy as jnp
from jax.experimental import pallas as pl
from jax.experimental.pallas import tpu as pltpu

Q = 1024
D = 32
V = 100000
NEG_INV_T_LOG2E = -10.0 * 1.4426950408889634
VBLK = 4000


def _soft_min_kernel(q_ref, k_ref, score_ref, thresh_ref, vocab_ref,
                     qn_ref, colf_ref, macc_ref, z_ref, w_ref, iacc_ref,
                     *, nblk, vblk):
    i = pl.program_id(0)

    @pl.when(i == 0)
    def _init():
        q = q_ref[...]
        qnorm = jnp.sqrt(jnp.sum(q * q, axis=-1, keepdims=True))
        qn_ref[...] = q / (qnorm + 1e-8)
        colf_ref[...] = jax.lax.broadcasted_iota(
            jnp.int32, (1, vblk), 1).astype(jnp.float32)
        macc_ref[...] = jnp.full((Q, vblk), jnp.inf, jnp.float32)
        z_ref[...] = jnp.zeros((Q, 1), jnp.float32)
        w_ref[...] = jnp.zeros((Q, 1), jnp.float32)
        iacc_ref[...] = jnp.zeros((Q, vblk), jnp.float32)

    k = k_ref[...]
    knorm = jnp.sqrt(jnp.sum(k * k, axis=-1, keepdims=True))
    kn = k / (knorm + 1e-8)
    sim = jax.lax.dot_general(
        qn_ref[...], kn, (((1,), (1,)), ((), ())),
        preferred_element_type=jnp.float32)
    dist = 1.0 - sim

    e = jnp.exp2(dist * NEG_INV_T_LOG2E)
    z_ref[...] += jnp.sum(e, axis=1, keepdims=True)
    w_ref[...] += jnp.sum(dist * e, axis=1, keepdims=True)

    cg = colf_ref[...] + (i * vblk).astype(jnp.float32)
    macc = macc_ref[...]
    iacc_ref[...] = jnp.where(dist < macc, cg, iacc_ref[...])
    macc_ref[...] = jnp.minimum(macc, dist)

    @pl.when(i == nblk - 1)
    def _finish():
        value = w_ref[...] / z_ref[...]
        score_ref[...] = value
        t = 1.0 - 2.0 * value
        celu = jnp.where(t > 0.0, t, jnp.exp(t) - 1.0)
        thresh_ref[...] = (celu + 1.0) * 0.5
        macc = macc_ref[...]
        bm = jnp.min(macc, axis=1, keepdims=True)
        cand = jnp.where(macc <= bm, iacc_ref[...], float(V))
        vocab_ref[...] = jnp.min(cand, axis=1,
                                 keepdims=True).astype(jnp.int32)


@jax.jit
def kernel(queries, keys):
    nblk = V // VBLK
    out = pl.pallas_call(
        functools.partial(_soft_min_kernel, nblk=nblk, vblk=VBLK),
        grid=(nblk,),
        in_specs=[
            pl.BlockSpec((Q, D), lambda i: (0, 0)),
            pl.BlockSpec((VBLK, D), lambda i: (i, 0)),
        ],
        out_specs=[
            pl.BlockSpec((Q, 1), lambda i: (0, 0)),
            pl.BlockSpec((Q, 1), lambda i: (0, 0)),
            pl.BlockSpec((Q, 1), lambda i: (0, 0)),
        ],
        out_shape=[
            jax.ShapeDtypeStruct((Q, 1), jnp.float32),
            jax.ShapeDtypeStruct((Q, 1), jnp.float32),
            jax.ShapeDtypeStruct((Q, 1), jnp.int32),
        ],
        scratch_shapes=[
            pltpu.VMEM((Q, D), jnp.float32),
            pltpu.VMEM((1, VBLK), jnp.float32),
            pltpu.VMEM((Q, VBLK), jnp.float32),
            pltpu.VMEM((Q, 1), jnp.float32),
            pltpu.VMEM((Q, 1), jnp.float32),
            pltpu.VMEM((Q, VBLK), jnp.float32),
        ],
    )(queries, keys)
    score, thresh, vocab = out
    return score.reshape(-1), thresh.reshape(-1), vocab.reshape(-1)
